# baseline (speedup 1.0000x reference)
import functools

import jax
import jax.numpy as jnp
from jax import lax
from jax.experimental import pallas as pl
from jax.experimental.pallas import tpu as pltpu
from jax.experimental.pallas import tpu_sc as plsc

_SCALE = 8.0
_LANES = 16


@functools.cache
def _make_gather(V, D, H, B):
    info = plsc.get_sparse_core_info()
    NC, NS = info.num_cores, info.num_subcores
    NW = NC * NS
    BT = B // 128
    assert BT == NW and D == 64
    DT = D // 8

    mesh = plsc.VectorSubcoreMesh(core_axis_name="c", subcore_axis_name="s")

    @functools.partial(
        pl.kernel,
        mesh=mesh,
        out_type=jax.ShapeDtypeStruct((H, DT, BT, 8, 128), jnp.float32),
        name="emb_gather_sc",
        scratch_types=[
            pltpu.VMEM((H, 128), jnp.int32),
            pltpu.VMEM((128, D), jnp.float32),
            pltpu.VMEM((128, D), jnp.float32),
            pltpu.VMEM((DT, 1, 8, 128), jnp.float32),
            pltpu.VMEM((DT, 1, 8, 128), jnp.float32),
            pltpu.SemaphoreType.DMA,
            pltpu.SemaphoreType.DMA,
            pltpu.SemaphoreType.DMA,
            pltpu.SemaphoreType.DMA,
        ],
        compiler_params=pltpu.CompilerParams(
            use_tc_tiling_on_sc=False, needs_layout_passes=False),
    )
    def gather_kernel(table_hbm, idx_hbm, out_hbm,
                      idx_v, a0, a1, b0, b1, gs0, gs1, os0, os1):
        A = (a0, a1)
        Bb = (b0, b1)
        gsem = (gs0, gs1)
        osem = (os0, os1)
        wid = lax.axis_index("s") * NC + lax.axis_index("c")

        pltpu.sync_copy(idx_hbm.at[:, pl.ds(wid * 128, 128)], idx_v)

        iot = lax.iota(jnp.int32, _LANES)
        hi = lax.shift_right_logical(iot, 3)
        zer = iot * 0
        ds_v = iot & 7

        def start_gather(h, buf):
            pltpu.async_copy(
                table_hbm.at[idx_v.at[h]], A[buf], gsem[buf])

        def wait_gather(h, buf):
            pltpu.make_async_copy(
                table_hbm.at[idx_v.at[h]], A[buf], gsem[buf]).wait()

        def start_out(h, buf):
            pltpu.async_copy(
                Bb[buf], out_hbm.at[h, :, pl.ds(wid, 1)], osem[buf])

        def wait_out(h, buf):
            pltpu.make_async_copy(
                Bb[buf], out_hbm.at[h, :, pl.ds(wid, 1)], osem[buf]).wait()

        start_gather(0, 0)
        start_gather(1, 1)

        def pair_body(p, carry):
            for buf in range(2):
                h = p * 2 + buf
                wait_gather(h, buf)

                @pl.when(h >= 2)
                def _():
                    wait_out(h - 2, buf)

                a = A[buf]
                bv = Bb[buf]

                @plsc.parallel_loop(0, 128, unroll=2)
                def _(bl):
                    bl_v = zer + bl
                    for g in range(D // _LANES):
                        v = a[bl, pl.ds(g * _LANES, _LANES)] * _SCALE
                        plsc.store_scatter(
                            bv, [hi + 2 * g, zer, ds_v, bl_v], v)

                start_out(h, buf)

                @pl.when(h + 2 < H)
                def _():
                    start_gather(h + 2, buf)

            return carry

        lax.fori_loop(0, H // 2, pair_body, 0)
        wait_out(H - 2, 0)
        wait_out(H - 1, 1)

    return gather_kernel


def kernel(x, table):
    B, H = x.shape
    V, D = table.shape
    xT = jnp.transpose(x).astype(jnp.int32)
    img = _make_gather(V, D, H, B)(table, xT)
    out = img.transpose(2, 4, 0, 1, 3).reshape(B, H, D)
    return out
